# baseline (speedup 1.0000x reference)
import functools

import numpy as np
import jax
import jax.numpy as jnp
from jax import lax
from jax.experimental import pallas as pl
from jax.experimental.pallas import tpu as pltpu
from jax.experimental.pallas import tpu_sc as plsc

NC = 2
NS = 16
NW = NC * NS
L = 16

B, C, S = 16, 128, 4096
NP = 21
NPP = 22
ROWS = B * C
RPW = ROWS // NW
T = 4
GROUPS = RPW // T
NV = S // L
NB = 2048
SG = T * S

_POS_XOR = np.int32(0x7FFFFFFF)
_M11 = np.int32(0x7FF)
_M10 = np.int32(0x3FF)


def _desc_key(u):
  negm = lax.shift_right_arithmetic(u, 31)
  return lax.bitwise_xor(u, lax.bitwise_and(lax.bitwise_not(negm), _POS_XOR))


def _body(x_hbm, w_hbm, n_hbm, out_hbm, perm_hbm, *scr):
  xg, kbuf0, kbuf1, ibuf0, ibuf1 = scr[:5]
  hA = scr[5:5 + T]
  hB = scr[5 + T:5 + 2 * T]
  wv, nv, ilbuf, frbuf, fcbuf, outv, sem_in, sem_out = scr[5 + 2 * T:]

  cid = lax.axis_index("c")
  sid = lax.axis_index("s")
  wid = sid * NC + cid
  row0 = wid * RPW
  grp0 = wid * GROUPS
  b = row0 // C
  c0 = row0 % C

  pltpu.sync_copy(n_hbm, nv)
  pltpu.sync_copy(w_hbm.at[pl.ds(c0 * NPP, RPW * NPP)], wv)

  lane = lax.iota(jnp.int32, L)
  n_b = jnp.sum(jnp.where(lane == b, nv[...], 0))
  total = jnp.maximum(n_b.astype(jnp.float32) - 1.0, 1.0)

  def prec(j, _):
    o = pl.multiple_of(j * L, L)
    sidx = lane + o
    t = jnp.minimum(sidx.astype(jnp.float32) / total, 1.0) * 20.0
    il = t.astype(jnp.int32)
    ilbuf[pl.ds(o, L)] = il
    fr = t - il.astype(jnp.float32)
    frbuf[pl.ds(o, L)] = fr
    fcbuf[pl.ds(o, L)] = 1.0 - fr
    return 0
  lax.fori_loop(0, NV, prec, 0)

  zeros16 = jnp.zeros((L,), jnp.int32)

  def hists_zero(hs, nbins):
    def z(j, _):
      o = pl.ds(pl.multiple_of(j * L, L), L)
      for h in hs:
        h[o] = zeros16
      return 0
    lax.fori_loop(0, nbins // L, z, 0)

  def hists_scan(hs, nbins, zero_hs=None):
    idx15 = jnp.full((L,), L - 1, jnp.int32)

    def sc(j, runs):
      o = pl.ds(pl.multiple_of(j * L, L), L)
      vs = [h[o] for h in hs]
      cs = [plsc.cumsum(v) for v in vs]
      for h, v, cum, run in zip(hs, vs, cs, runs):
        h[o] = cum - v + run
      if zero_hs is not None:
        for h in zero_hs:
          h[o] = zeros16
      return tuple(run + jnp.take_along_axis(cum, idx15, axis=0)
                   for run, cum in zip(runs, cs))
    lax.fori_loop(0, nbins // L, sc,
                  tuple(jnp.full((L,), t * S - 1, jnp.int32)
                        for t in range(T)), unroll=2)

  def tf_stage(j, xbase):
    o = pl.multiple_of(j * L, L)
    us = [plsc.bitcast(xg[pl.ds(pl.multiple_of(xbase + o + t * S, L), L)],
                       jnp.int32)
          for t in range(T)]
    ks = [_desc_key(u) for u in us]
    for t in range(T):
      kbuf0[pl.ds(o + t * S, L)] = ks[t]
    ds = [lax.bitwise_and(k, _M11) for k in ks]
    ones = jnp.full((L,), 1, jnp.int32)
    for t in range(T):
      plsc.addupdate_scatter(hA[t], [ds[t]], ones)

  pltpu.sync_copy(x_hbm.at[grp0], xg.at[pl.ds(0, SG)])
  hists_zero(hA, NB)

  def tf0(j, _):
    tf_stage(j, 0)
    return 0
  lax.fori_loop(0, NV, tf0, 0, unroll=2)

  def do_group(rr, _):
    nxt = jnp.minimum(rr + 1, GROUPS - 1)
    xbase_nxt = lax.rem(rr + 1, 2) * SG
    in_copy = pltpu.async_copy(
        x_hbm.at[grp0 + nxt],
        xg.at[pl.ds(pl.multiple_of(xbase_nxt, SG), SG)], sem_in)

    hists_scan(hA, NB, zero_hs=hB)

    def permute(ksrc, isrc, kdst, idst, shift, mask, hsrc,
                next_shift, next_mask, hdst):
      def body(j, _):
        o = pl.multiple_of(j * L, L)
        ks = [ksrc[pl.ds(o + t * S, L)] for t in range(T)]
        if isrc is None:
          iv0 = lane + o
          ivs = [iv0] * T
        else:
          ivs = [isrc[pl.ds(o + t * S, L)] for t in range(T)]
        ds = [lax.bitwise_and(lax.shift_right_logical(k, shift), mask)
              for k in ks]
        sc = [plsc.scan_count(d) for d in ds]
        offs = [plsc.load_gather(h, [d]) for h, d in zip(hsrc, ds)]
        poss = [off + cnt for off, (cnt, _) in zip(offs, sc)]
        if kdst is not None:
          for t in range(T):
            plsc.store_scatter(kdst, [poss[t]], ks[t])
        for t in range(T):
          plsc.store_scatter(idst, [poss[t]], ivs[t])
        for t in range(T):
          plsc.store_scatter(hsrc[t], [ds[t]], poss[t], mask=sc[t][1])
        if next_shift is not None:
          d2 = [lax.bitwise_and(lax.shift_right_logical(k, next_shift),
                                next_mask) for k in ks]
          ones = jnp.full((L,), 1, jnp.int32)
          for t in range(T):
            plsc.addupdate_scatter(hdst[t], [d2[t]], ones)
        return 0
      lax.fori_loop(0, NV, body, 0, unroll=2)

    permute(kbuf0, None, kbuf1, ibuf1, 0, _M11, hA, 11, _M11, hB)
    hists_scan(hB, NB, zero_hs=hA)
    permute(kbuf1, ibuf1, kbuf0, ibuf0, 11, _M11, hB, 22, _M10, hA)
    hists_scan(hA, 1024)
    permute(kbuf0, ibuf0, kbuf1, ibuf1, 22, _M10, hA, None, None, None)

    out_copy = pltpu.async_copy(ibuf1, perm_hbm.at[grp0 + rr], sem_out)

    hists_zero(hA, 1024)
    in_copy.wait()

    wbs = [(T * rr + t) * NPP for t in range(T)]

    def fs(j, accs):
      o = pl.multiple_of(j * L, L)
      ks = [kbuf1[pl.ds(o + t * S, L)] for t in range(T)]
      il = ilbuf[pl.ds(o, L)]
      fr = frbuf[pl.ds(o, L)]
      fc = fcbuf[pl.ds(o, L)]
      ilp = il + 1
      vs = [plsc.bitcast(_desc_key(k), jnp.float32) for k in ks]
      wls = [plsc.load_gather(wv, [il + wb]) for wb in wbs]
      wrs = [plsc.load_gather(wv, [ilp + wb]) for wb in wbs]
      tf_stage(j, xbase_nxt)
      return tuple(acc + v * (fc * wl + fr * wr)
                   for acc, v, wl, wr in zip(accs, vs, wls, wrs))
    accs = lax.fori_loop(0, NV, fs,
                         tuple(jnp.zeros((L,), jnp.float32)
                               for _ in range(T)), unroll=2)
    vals = [jnp.sum(a) for a in accs]
    sel = vals[T - 1]
    for t in range(T - 2, -1, -1):
      sel = jnp.where(lane == t, vals[t], sel)
    plsc.store_scatter(outv, [jnp.minimum(lane, T - 1) + T * rr], sel,
                       mask=lane < T)

    out_copy.wait()
    return 0

  lax.fori_loop(0, GROUPS, do_group, 0)
  pltpu.sync_copy(outv, out_hbm.at[pl.ds(row0, RPW)])


@jax.jit
def _fspool_sc(x2, wflat, n):
  run = pl.kernel(
      _body,
      out_type=(
          jax.ShapeDtypeStruct((ROWS,), jnp.float32),
          jax.ShapeDtypeStruct((ROWS // T, SG), jnp.int32),
      ),
      mesh=plsc.VectorSubcoreMesh(core_axis_name="c", subcore_axis_name="s"),
      scratch_types=(
          [
              pltpu.VMEM((2 * SG,), jnp.float32),
              pltpu.VMEM((SG,), jnp.int32),
              pltpu.VMEM((SG,), jnp.int32),
              pltpu.VMEM((SG,), jnp.int32),
              pltpu.VMEM((SG,), jnp.int32),
          ]
          + [pltpu.VMEM((NB,), jnp.int32) for _ in range(2 * T)]
          + [
              pltpu.VMEM((RPW * NPP,), jnp.float32),
              pltpu.VMEM((B,), jnp.int32),
              pltpu.VMEM((S,), jnp.int32),
              pltpu.VMEM((S,), jnp.float32),
              pltpu.VMEM((S,), jnp.float32),
              pltpu.VMEM((RPW,), jnp.float32),
              pltpu.SemaphoreType.DMA,
              pltpu.SemaphoreType.DMA,
          ]
      ),
      compiler_params=pltpu.CompilerParams(needs_layout_passes=False),
  )
  return run(x2, wflat, n)


def kernel(x, W, n):
  x2 = x.reshape(ROWS // T, SG)
  wpad = jnp.concatenate([W, W[:, -1:]], axis=1).reshape(-1)
  out_flat, perm2 = _fspool_sc(x2, wpad, n.astype(jnp.int32))
  return out_flat.reshape(B, C), perm2.reshape(B, C, S)


# speedup vs baseline: 1.0348x; 1.0348x over previous
import functools

import numpy as np
import jax
import jax.numpy as jnp
from jax import lax
from jax.experimental import pallas as pl
from jax.experimental.pallas import tpu as pltpu
from jax.experimental.pallas import tpu_sc as plsc

NC = 2
NS = 16
NW = NC * NS
L = 16

B, C, S = 16, 128, 4096
NP = 21
NPP = 22
ROWS = B * C
RPW = ROWS // NW
T = 4
GROUPS = RPW // T
NV = S // L
NB = 2048
SG = T * S

_POS_XOR = np.int32(0x7FFFFFFF)
_M11 = np.int32(0x7FF)
_M10 = np.int32(0x3FF)


def _desc_key(u):
  negm = lax.shift_right_arithmetic(u, 31)
  return lax.bitwise_xor(u, lax.bitwise_and(lax.bitwise_not(negm), _POS_XOR))


def _body(x_hbm, w_hbm, n_hbm, out_hbm, perm_hbm, *scr):
  xg, kbuf0, kbuf1, ibuf0, ibuf1 = scr[:5]
  hA = scr[5:5 + T]
  hB = scr[5 + T:5 + 2 * T]
  wv, nv, ilbuf, frbuf, fcbuf, outv, sem_in, sem_out = scr[5 + 2 * T:]

  cid = lax.axis_index("c")
  sid = lax.axis_index("s")
  wid = sid * NC + cid
  row0 = wid * RPW
  grp0 = wid * GROUPS
  b = row0 // C
  c0 = row0 % C

  pltpu.sync_copy(n_hbm, nv)
  pltpu.sync_copy(w_hbm.at[pl.ds(c0 * NPP, RPW * NPP)], wv)

  lane = lax.iota(jnp.int32, L)
  n_b = jnp.sum(jnp.where(lane == b, nv[...], 0))
  total = jnp.maximum(n_b.astype(jnp.float32) - 1.0, 1.0)

  def prec(j, _):
    o = pl.multiple_of(j * L, L)
    sidx = lane + o
    t = jnp.minimum(sidx.astype(jnp.float32) / total, 1.0) * 20.0
    il = t.astype(jnp.int32)
    ilbuf[pl.ds(o, L)] = il
    fr = t - il.astype(jnp.float32)
    frbuf[pl.ds(o, L)] = fr
    fcbuf[pl.ds(o, L)] = 1.0 - fr
    return 0
  lax.fori_loop(0, NV, prec, 0)

  zeros16 = jnp.zeros((L,), jnp.int32)

  def hists_zero(hs, nbins):
    def z(j, _):
      o = pl.ds(pl.multiple_of(j * L, L), L)
      for h in hs:
        h[o] = zeros16
      return 0
    lax.fori_loop(0, nbins // L, z, 0)

  def hists_scan(hs, nbins, zero_hs=None):
    idx15 = jnp.full((L,), L - 1, jnp.int32)

    def sc(j, runs):
      o = pl.ds(pl.multiple_of(j * L, L), L)
      vs = [h[o] for h in hs]
      cs = [plsc.cumsum(v) for v in vs]
      for h, v, cum, run in zip(hs, vs, cs, runs):
        h[o] = cum - v + run
      if zero_hs is not None:
        for h in zero_hs:
          h[o] = zeros16
      return tuple(run + jnp.take_along_axis(cum, idx15, axis=0)
                   for run, cum in zip(runs, cs))
    lax.fori_loop(0, nbins // L, sc,
                  tuple(jnp.full((L,), t * S - 1, jnp.int32)
                        for t in range(T)))

  def tf_stage(j, xbase):
    o = pl.multiple_of(j * L, L)
    us = [plsc.bitcast(xg[pl.ds(pl.multiple_of(xbase + o + t * S, L), L)],
                       jnp.int32)
          for t in range(T)]
    ks = [_desc_key(u) for u in us]
    for t in range(T):
      kbuf0[pl.ds(o + t * S, L)] = ks[t]
    ds = [lax.bitwise_and(k, _M11) for k in ks]
    ones = jnp.full((L,), 1, jnp.int32)
    for t in range(T):
      plsc.addupdate_scatter(hA[t], [ds[t]], ones)

  pltpu.sync_copy(x_hbm.at[grp0], xg.at[pl.ds(0, SG)])
  hists_zero(hA, NB)

  def tf0(j, _):
    tf_stage(j, 0)
    return 0
  lax.fori_loop(0, NV, tf0, 0, unroll=2)

  def do_group(rr, _):
    nxt = jnp.minimum(rr + 1, GROUPS - 1)
    xbase_nxt = lax.rem(rr + 1, 2) * SG
    in_copy = pltpu.async_copy(
        x_hbm.at[grp0 + nxt],
        xg.at[pl.ds(pl.multiple_of(xbase_nxt, SG), SG)], sem_in)

    hists_scan(hA, NB, zero_hs=hB)

    def permute(ksrc, isrc, kdst, idst, shift, mask, hsrc,
                next_shift, next_mask, hdst):
      def body(j, _):
        o = pl.multiple_of(j * L, L)
        ks = [ksrc[pl.ds(o + t * S, L)] for t in range(T)]
        if isrc is None:
          iv0 = lane + o
          ivs = [iv0] * T
        else:
          ivs = [isrc[pl.ds(o + t * S, L)] for t in range(T)]
        ds = [lax.bitwise_and(lax.shift_right_logical(k, shift), mask)
              for k in ks]
        sc = [plsc.scan_count(d) for d in ds]
        offs = [plsc.load_gather(h, [d]) for h, d in zip(hsrc, ds)]
        poss = [off + cnt for off, (cnt, _) in zip(offs, sc)]
        if kdst is not None:
          for t in range(T):
            plsc.store_scatter(kdst, [poss[t]], ks[t])
        for t in range(T):
          plsc.store_scatter(idst, [poss[t]], ivs[t])
        for t in range(T):
          plsc.store_scatter(hsrc[t], [ds[t]], poss[t], mask=sc[t][1])
        if next_shift is not None:
          d2 = [lax.bitwise_and(lax.shift_right_logical(k, next_shift),
                                next_mask) for k in ks]
          ones = jnp.full((L,), 1, jnp.int32)
          for t in range(T):
            plsc.addupdate_scatter(hdst[t], [d2[t]], ones)
        return 0
      lax.fori_loop(0, NV, body, 0, unroll=2)

    permute(kbuf0, None, kbuf1, ibuf1, 0, _M11, hA, 11, _M11, hB)
    hists_scan(hB, NB, zero_hs=hA)
    permute(kbuf1, ibuf1, kbuf0, ibuf0, 11, _M11, hB, 22, _M10, hA)
    hists_scan(hA, 1024)
    permute(kbuf0, ibuf0, kbuf1, ibuf1, 22, _M10, hA, None, None, None)

    out_copy = pltpu.async_copy(ibuf1, perm_hbm.at[grp0 + rr], sem_out)

    hists_zero(hA, 1024)
    in_copy.wait()

    wbs = [(T * rr + t) * NPP for t in range(T)]

    def fs(j, accs):
      o = pl.multiple_of(j * L, L)
      ks = [kbuf1[pl.ds(o + t * S, L)] for t in range(T)]
      il = ilbuf[pl.ds(o, L)]
      fr = frbuf[pl.ds(o, L)]
      fc = fcbuf[pl.ds(o, L)]
      ilp = il + 1
      vs = [plsc.bitcast(_desc_key(k), jnp.float32) for k in ks]
      wls = [plsc.load_gather(wv, [il + wb]) for wb in wbs]
      wrs = [plsc.load_gather(wv, [ilp + wb]) for wb in wbs]
      tf_stage(j, xbase_nxt)
      return tuple(acc + v * (fc * wl + fr * wr)
                   for acc, v, wl, wr in zip(accs, vs, wls, wrs))
    accs = lax.fori_loop(0, NV, fs,
                         tuple(jnp.zeros((L,), jnp.float32)
                               for _ in range(T)))
    vals = [jnp.sum(a) for a in accs]
    sel = vals[T - 1]
    for t in range(T - 2, -1, -1):
      sel = jnp.where(lane == t, vals[t], sel)
    plsc.store_scatter(outv, [jnp.minimum(lane, T - 1) + T * rr], sel,
                       mask=lane < T)

    out_copy.wait()
    return 0

  lax.fori_loop(0, GROUPS, do_group, 0)
  pltpu.sync_copy(outv, out_hbm.at[pl.ds(row0, RPW)])


@jax.jit
def _fspool_sc(x2, wflat, n):
  run = pl.kernel(
      _body,
      out_type=(
          jax.ShapeDtypeStruct((ROWS,), jnp.float32),
          jax.ShapeDtypeStruct((ROWS // T, SG), jnp.int32),
      ),
      mesh=plsc.VectorSubcoreMesh(core_axis_name="c", subcore_axis_name="s"),
      scratch_types=(
          [
              pltpu.VMEM((2 * SG,), jnp.float32),
              pltpu.VMEM((SG,), jnp.int32),
              pltpu.VMEM((SG,), jnp.int32),
              pltpu.VMEM((SG,), jnp.int32),
              pltpu.VMEM((SG,), jnp.int32),
          ]
          + [pltpu.VMEM((NB,), jnp.int32) for _ in range(2 * T)]
          + [
              pltpu.VMEM((RPW * NPP,), jnp.float32),
              pltpu.VMEM((B,), jnp.int32),
              pltpu.VMEM((S,), jnp.int32),
              pltpu.VMEM((S,), jnp.float32),
              pltpu.VMEM((S,), jnp.float32),
              pltpu.VMEM((RPW,), jnp.float32),
              pltpu.SemaphoreType.DMA,
              pltpu.SemaphoreType.DMA,
          ]
      ),
      compiler_params=pltpu.CompilerParams(needs_layout_passes=False),
  )
  return run(x2, wflat, n)


def kernel(x, W, n):
  x2 = x.reshape(ROWS // T, SG)
  wpad = jnp.concatenate([W, W[:, -1:]], axis=1).reshape(-1)
  out_flat, perm2 = _fspool_sc(x2, wpad, n.astype(jnp.int32))
  return out_flat.reshape(B, C), perm2.reshape(B, C, S)
